# baseline (device time: 13739 ns/iter reference)
import jax
import jax.numpy as jnp
from jax import lax
from jax.experimental import pallas as pl
from jax.experimental.pallas import tpu as pltpu

M = 1024
D = 512
HALF = M // 2
QTR = HALF // 2
K = 4
CH = QTR // K


def kernel(partial, gamma):
    def body(partial_ref, gamma_ref, out_ref,
             raw_send, local_buf, send_buf, yrecv_buf, xrecv_buf, gamma_buf,
             ld_sems, ysend_sems, yrecv_sems, xsend_sems, xrecv_sems):
        my_x = lax.axis_index("x")
        my_y = lax.axis_index("y")
        ynbr = (my_x, 1 - my_y)
        xnbr = (1 - my_x, my_y)

        send_start = (1 - my_y) * HALF + my_x * QTR
        my_start = my_y * HALF
        ld_send = []
        for k in range(K):
            dma = pltpu.make_async_copy(
                partial_ref.at[0, pl.ds(send_start + k * CH, CH), :],
                raw_send.at[k],
                ld_sems.at[k],
            )
            dma.start()
            ld_send.append(dma)
        dma_local = pltpu.make_async_copy(
            partial_ref.at[0, pl.ds(my_start, HALF), :], local_buf,
            ld_sems.at[K],
        )
        dma_local.start()
        dma_gamma = pltpu.make_async_copy(
            gamma_ref, gamma_buf, ld_sems.at[K + 1],
        )
        dma_gamma.start()

        barrier_sem = pltpu.get_barrier_semaphore()
        for nbr in (ynbr, xnbr):
            pl.semaphore_signal(
                barrier_sem, inc=1,
                device_id=nbr, device_id_type=pl.DeviceIdType.MESH,
            )
        for k in range(K):
            ld_send[k].wait()
            send_buf[k] = raw_send[k].astype(jnp.bfloat16)
        pl.semaphore_wait(barrier_sem, 2)

        y_rdmas = []
        for k in range(K):
            r = pltpu.make_async_remote_copy(
                src_ref=send_buf.at[k],
                dst_ref=yrecv_buf.at[k],
                send_sem=ysend_sems.at[k],
                recv_sem=yrecv_sems.at[k],
                device_id=ynbr,
                device_id_type=pl.DeviceIdType.MESH,
            )
            r.start()
            y_rdmas.append(r)

        dma_local.wait()
        dma_gamma.wait()

        def rmsnorm(addend_bf16, row0):
            y = local_buf[pl.ds(row0, CH), :] + addend_bf16.astype(jnp.float32)
            ms = jnp.mean(y * y, axis=-1, keepdims=True)
            return (y * lax.rsqrt(ms + 1e-6) * gamma_buf[...]).astype(
                jnp.bfloat16
            )

        my_q = my_x * QTR
        x_rdmas = []
        for k in range(K):
            y_rdmas[k].wait_recv()
            r = pltpu.make_async_remote_copy(
                src_ref=yrecv_buf.at[k],
                dst_ref=xrecv_buf.at[k],
                send_sem=xsend_sems.at[k],
                recv_sem=xrecv_sems.at[k],
                device_id=xnbr,
                device_id_type=pl.DeviceIdType.MESH,
            )
            r.start()
            x_rdmas.append(r)
            out_ref[pl.ds(my_q + k * CH, CH), :] = rmsnorm(
                yrecv_buf[k], my_q + k * CH
            )

        other_q = (1 - my_x) * QTR
        for k in range(K):
            x_rdmas[k].wait_recv()
            out_ref[pl.ds(other_q + k * CH, CH), :] = rmsnorm(
                xrecv_buf[k], other_q + k * CH
            )

        for k in range(K):
            y_rdmas[k].wait_send()
            x_rdmas[k].wait_send()

    return pl.pallas_call(
        body,
        out_shape=jax.ShapeDtypeStruct((HALF, D), jnp.bfloat16),
        in_specs=[
            pl.BlockSpec(memory_space=pltpu.MemorySpace.HBM),
            pl.BlockSpec(memory_space=pltpu.MemorySpace.HBM),
        ],
        out_specs=pl.BlockSpec(memory_space=pltpu.VMEM),
        scratch_shapes=[
            pltpu.VMEM((K, CH, D), jnp.float32),
            pltpu.VMEM((HALF, D), jnp.float32),
            pltpu.VMEM((K, CH, D), jnp.bfloat16),
            pltpu.VMEM((K, CH, D), jnp.bfloat16),
            pltpu.VMEM((K, CH, D), jnp.bfloat16),
            pltpu.VMEM((1, D), jnp.float32),
            pltpu.SemaphoreType.DMA((K + 2,)),
            pltpu.SemaphoreType.DMA((K,)),
            pltpu.SemaphoreType.DMA((K,)),
            pltpu.SemaphoreType.DMA((K,)),
            pltpu.SemaphoreType.DMA((K,)),
        ],
        compiler_params=pltpu.CompilerParams(collective_id=0),
    )(partial, gamma.reshape(1, D))


# device time: 12723 ns/iter; 1.0799x vs baseline; 1.0799x over previous
import jax
import jax.numpy as jnp
from jax import lax
from jax.experimental import pallas as pl
from jax.experimental.pallas import tpu as pltpu

M = 1024
D = 512
HALF = M // 2
K = 4
CH = HALF // K


def kernel(partial, gamma):
    def body(partial_ref, gamma_ref, out_ref,
             raw_send, local_buf, send_buf, recv_buf, gamma_buf,
             ld_sems, send_sems, recv_sems):
        my_x = lax.axis_index("x")
        my_y = lax.axis_index("y")
        nbr = (my_x, 1 - my_y)

        nbr_start = (1 - my_y) * HALF
        my_start = my_y * HALF
        ld_send = []
        for k in range(K):
            dma = pltpu.make_async_copy(
                partial_ref.at[0, pl.ds(nbr_start + k * CH, CH), :],
                raw_send.at[k],
                ld_sems.at[k],
            )
            dma.start()
            ld_send.append(dma)
        dma_local = pltpu.make_async_copy(
            partial_ref.at[0, pl.ds(my_start, HALF), :], local_buf,
            ld_sems.at[K],
        )
        dma_local.start()
        dma_gamma = pltpu.make_async_copy(
            gamma_ref, gamma_buf, ld_sems.at[K + 1],
        )
        dma_gamma.start()

        barrier_sem = pltpu.get_barrier_semaphore()
        pl.semaphore_signal(
            barrier_sem, inc=1,
            device_id=nbr, device_id_type=pl.DeviceIdType.MESH,
        )
        for k in range(K):
            ld_send[k].wait()
            send_buf[k] = raw_send[k].astype(jnp.bfloat16)
        pl.semaphore_wait(barrier_sem, 1)

        rdmas = []
        for k in range(K):
            r = pltpu.make_async_remote_copy(
                src_ref=send_buf.at[k],
                dst_ref=recv_buf.at[k],
                send_sem=send_sems.at[k],
                recv_sem=recv_sems.at[k],
                device_id=nbr,
                device_id_type=pl.DeviceIdType.MESH,
            )
            r.start()
            rdmas.append(r)

        dma_local.wait()
        dma_gamma.wait()

        for k in range(K):
            rdmas[k].wait_recv()
            y = local_buf[pl.ds(k * CH, CH), :] + recv_buf[k].astype(jnp.float32)
            ms = jnp.mean(y * y, axis=-1, keepdims=True)
            out_ref[pl.ds(k * CH, CH), :] = (
                y * lax.rsqrt(ms + 1e-6) * gamma_buf[...]
            ).astype(jnp.bfloat16)

        for k in range(K):
            rdmas[k].wait_send()

    return pl.pallas_call(
        body,
        out_shape=jax.ShapeDtypeStruct((HALF, D), jnp.bfloat16),
        in_specs=[
            pl.BlockSpec(memory_space=pltpu.MemorySpace.HBM),
            pl.BlockSpec(memory_space=pltpu.MemorySpace.HBM),
        ],
        out_specs=pl.BlockSpec(memory_space=pltpu.VMEM),
        scratch_shapes=[
            pltpu.VMEM((K, CH, D), jnp.float32),
            pltpu.VMEM((HALF, D), jnp.float32),
            pltpu.VMEM((K, CH, D), jnp.bfloat16),
            pltpu.VMEM((K, CH, D), jnp.bfloat16),
            pltpu.VMEM((1, D), jnp.float32),
            pltpu.SemaphoreType.DMA((K + 2,)),
            pltpu.SemaphoreType.DMA((K,)),
            pltpu.SemaphoreType.DMA((K,)),
        ],
        compiler_params=pltpu.CompilerParams(collective_id=0),
    )(partial, gamma.reshape(1, D))
